# Initial kernel scaffold; baseline (speedup 1.0000x reference)
import jax
import jax.numpy as jnp
from jax.experimental import pallas as pl


def kernel(x, edge_index, W1, a_s1, a_d1, b1, W2, a_s2, a_d2, b2):
    raise NotImplementedError("write your pallas kernel here")


# speedup vs baseline: 12.7272x; 12.7272x over previous
import jax
import jax.numpy as jnp
from jax import lax
from jax.experimental import pallas as pl
from jax.experimental.pallas import tpu as pltpu
from jax.experimental.pallas import tpu_sc as plsc

N = 10000
NP = 10240
E = 320000
NT = 32
BUCK = 320
LCAP = 896
CF = 2000
C = 32
CGRP = C // 16
ACC1_W = 144
ACC2_W = 32

_F32 = jnp.float32
_I32 = jnp.int32


def _iota16():
    return lax.iota(_I32, 16)


def _splat_i32(v):
    return jnp.full((16,), v, dtype=_I32)


def _bcast(v, dtype):
    return lax.broadcast_in_dim(lax.convert_element_type(v, dtype), (16,), ())


_GATHER_DN = lax.GatherDimensionNumbers(
    offset_dims=(), collapsed_slice_dims=(0,), start_index_map=(0,))


def _take16(vec, idx_vec):
    return lax.gather(vec, idx_vec[:, None], _GATHER_DN, (1,),
                      mode=lax.GatherScatterMode.PROMISE_IN_BOUNDS)


def _maxsplat(v):
    it = _iota16()
    for k in (1, 2, 4, 8):
        v = jnp.maximum(v, _take16(v, jnp.bitwise_xor(it, k)))
    return v


def _tc_a_body(x_ref, w1_ref, asw_ref, adw_ref, h1_ref, aa_ref, m1_ref):
    h = jnp.dot(x_ref[...], w1_ref[...], preferred_element_type=_F32)
    h1_ref[...] = h
    asn = jnp.dot(h, asw_ref[...], preferred_element_type=_F32)
    adn = jnp.dot(h, adw_ref[...], preferred_element_type=_F32)
    aa_ref[...] = jnp.concatenate(
        [asn, adn, jnp.zeros((NP, 112), _F32)], axis=1)
    m8 = jnp.maximum(jnp.max(asn, axis=0) + jnp.max(adn, axis=0), 0.0)
    m1_ref[...] = jnp.concatenate([m8, jnp.zeros((8,), _F32)]).reshape(1, 16)


def _tc_a(xp, w1, asw, adw):
    return pl.pallas_call(
        _tc_a_body,
        out_shape=[
            jax.ShapeDtypeStruct((NP, 128), _F32),
            jax.ShapeDtypeStruct((NP, 128), _F32),
            jax.ShapeDtypeStruct((1, 16), _F32),
        ],
    )(xp, w1, asw, adw)


def _sc_b_body(esrc, edst, h1, aa1, m1,
               accs, lsrc, ldst, counts,
               ebuf_s, ebuf_d, list_s, list_d,
               aas, aad, hrows, acc, m1v, cntb, scrn,
               idxbuf, dbuf,
               sem1, sem2, sem3):
    wid = lax.axis_index("s") * 2 + lax.axis_index("c")
    base = wid * BUCK

    def _zacc(i, _):
        zf = jnp.zeros((16,), _F32)
        for j in range(ACC1_W // 16):
            acc[i, pl.ds(j * 16, 16)] = zf
        return 0
    lax.fori_loop(0, BUCK, _zacc, 0)

    pltpu.sync_copy(m1, m1v)

    def _vstep(j, curv):
        it = _iota16()
        sv = ebuf_s[pl.ds(j * 16, 16)]
        dv = ebuf_d[pl.ds(j * 16, 16)]
        m = (dv >= _bcast(base, _I32)) & (dv < _bcast(base + BUCK, _I32))
        off = jnp.where(m, jnp.minimum(curv, LCAP - 2), _splat_i32(LCAP - 1))
        plsc.store_scatter(list_s, [it, off], sv.astype(_F32))
        plsc.store_scatter(list_d, [it, off], dv.astype(_F32))
        return jnp.minimum(curv + m.astype(_I32), LCAP - 1)

    def _scan(ci, curv):
        pltpu.sync_copy(esrc.at[pl.ds(ci * CF, CF)], ebuf_s)
        pltpu.sync_copy(edst.at[pl.ds(ci * CF, CF)], ebuf_d)
        return lax.fori_loop(0, CF // 16, _vstep, curv)

    scrn[...] = _splat_i32(E // CF)
    curv = lax.fori_loop(0, scrn[...][0], _scan, _splat_i32(0))

    cntb[...] = curv
    pltpu.sync_copy(cntb, counts.at[wid])
    pltpu.sync_copy(list_s, lsrc.at[wid])
    pltpu.sync_copy(list_d, ldst.at[wid])

    scrn[...] = (_maxsplat(curv) + (CGRP - 1)) // CGRP
    nch = scrn[...][0]

    def _chunk(k, _):
        for j in range(CGRP):
            it = _iota16()
            cntv = cntb[...]
            o = k * CGRP + j
            ov = _bcast(o, _I32)
            validv = ov < cntv
            sviv = plsc.load_gather(list_s, [it, ov]).astype(_I32)
            dviv = plsc.load_gather(list_d, [it, ov]).astype(_I32)
            sviv = jnp.where(validv, sviv, 0)
            dviv = jnp.where(validv, dviv, _bcast(base, _I32))
            idxbuf[pl.ds(j * 16, 16)] = sviv
            dbuf[pl.ds(j * 16, 16)] = dviv
        cp1 = pltpu.async_copy(aa1.at[idxbuf], aas, sem1)
        cp2 = pltpu.async_copy(aa1.at[dbuf], aad, sem2)
        cp3 = pltpu.async_copy(h1.at[idxbuf], hrows, sem3)
        cp1.wait()
        cp2.wait()
        cp3.wait()

        for g in range(CGRP):
            it = _iota16()
            sh8 = jnp.minimum(it + 8, 15)
            col_d = it + 128
            mask8 = it < 8
            m1g = m1v[0, :]
            cntv = cntb[...]
            maskf = (_bcast(k * CGRP + g, _I32) < cntv).astype(_F32)
            dstv = dbuf[pl.ds(g * 16, 16)]
            dlocv = jnp.clip(dstv - _bcast(base, _I32), 0, BUCK - 1)
            for e in range(16):
                idx = g * 16 + e
                rs = aas[idx, pl.ds(0, 16)]
                rd = aad[idx, pl.ds(0, 16)]
                e8 = rs + _take16(rd, sh8)
                e8 = jnp.maximum(e8, 0.2 * e8)
                w = jnp.exp(e8 - m1g)
                w = w * _take16(maskf, _splat_i32(e))
                dspl = _take16(dlocv, _splat_i32(e))
                for h in range(8):
                    wh = _take16(w, _splat_i32(h))
                    hv = hrows[idx, pl.ds(16 * h, 16)]
                    plsc.addupdate_scatter(acc, [dspl, it + 16 * h], wh * hv)
                plsc.addupdate_scatter(acc, [dspl, col_d], w)
        return 0

    lax.fori_loop(0, nch, _chunk, 0)
    pltpu.sync_copy(acc, accs.at[wid])


def _sc_b(esrc, edst, h1, aa1, m1):
    mesh = plsc.VectorSubcoreMesh(core_axis_name="c", subcore_axis_name="s")
    kern = pl.kernel(
        _sc_b_body,
        out_type=[
            jax.ShapeDtypeStruct((NT, BUCK, ACC1_W), _F32),
            jax.ShapeDtypeStruct((NT, 16, LCAP), _F32),
            jax.ShapeDtypeStruct((NT, 16, LCAP), _F32),
            jax.ShapeDtypeStruct((NT, 16), _I32),
        ],
        mesh=mesh,
        scratch_types=[
            pltpu.VMEM((CF,), _I32),
            pltpu.VMEM((CF,), _I32),
            pltpu.VMEM((16, LCAP), _F32),
            pltpu.VMEM((16, LCAP), _F32),
            pltpu.VMEM((C, 128), _F32),
            pltpu.VMEM((C, 128), _F32),
            pltpu.VMEM((C, 128), _F32),
            pltpu.VMEM((BUCK, ACC1_W), _F32),
            pltpu.VMEM((1, 16), _F32),
            pltpu.VMEM((16,), _I32),
            pltpu.VMEM((16,), _I32),
            pltpu.VMEM((C,), _I32),
            pltpu.VMEM((C,), _I32),
            pltpu.SemaphoreType.DMA,
            pltpu.SemaphoreType.DMA,
            pltpu.SemaphoreType.DMA,
        ],
        compiler_params=pltpu.CompilerParams(needs_layout_passes=False),
    )
    return kern(esrc, edst, h1, aa1, m1)


def _tc_c_body(accs_ref, w2_ref, e8t_ref, b1_ref, asv_ref, adv_ref, h2x_ref):
    a = accs_ref[...]
    hs = a[:, :128]
    dn = a[:, 128:136]
    dninv = 1.0 / (dn + 1e-16)
    x2 = hs * jnp.dot(dninv, e8t_ref[...], preferred_element_type=_F32)
    x2 = x2 + b1_ref[...]
    x2 = jnp.where(x2 > 0, x2, jnp.exp(jnp.minimum(x2, 0.0)) - 1.0)
    h2 = jnp.dot(x2, w2_ref[...], preferred_element_type=_F32)
    as2 = jnp.sum(h2 * asv_ref[...], axis=1, keepdims=True)
    ad2 = jnp.sum(h2 * adv_ref[...], axis=1, keepdims=True)
    m2 = jnp.maximum(jnp.max(as2) + jnp.max(ad2), 0.0)
    m2b = jnp.full((NP, 1), m2, _F32)
    pad = jnp.zeros((NP, 109), _F32)
    h2x_ref[...] = jnp.concatenate([h2, as2, ad2, m2b, pad], axis=1)


def _tc_c(accs_flat, w2, e8t, b1r, asv, adv):
    return pl.pallas_call(
        _tc_c_body,
        out_shape=jax.ShapeDtypeStruct((NP, 128), _F32),
    )(accs_flat, w2, e8t, b1r, asv, adv)


def _sc_d_body(lsrc, ldst, counts, h2x,
               accs2,
               list_s, list_d, own, rows, acc2, cntb, scrn,
               idxbuf, dbuf,
               sem1):
    wid = lax.axis_index("s") * 2 + lax.axis_index("c")
    base = wid * BUCK

    def _zacc(i, _):
        zf = jnp.zeros((16,), _F32)
        for j in range(ACC2_W // 16):
            acc2[i, pl.ds(j * 16, 16)] = zf
        return 0
    lax.fori_loop(0, BUCK, _zacc, 0)

    pltpu.sync_copy(counts.at[wid], cntb)
    pltpu.sync_copy(lsrc.at[wid], list_s)
    pltpu.sync_copy(ldst.at[wid], list_d)
    pltpu.sync_copy(h2x.at[pl.ds(base, BUCK)], own)

    scrn[...] = (_maxsplat(cntb[...]) + (CGRP - 1)) // CGRP
    nch = scrn[...][0]

    def _chunk(k, _):
        for j in range(CGRP):
            it = _iota16()
            cntv = cntb[...]
            o = k * CGRP + j
            ov = _bcast(o, _I32)
            validv = ov < cntv
            sviv = plsc.load_gather(list_s, [it, ov]).astype(_I32)
            dviv = plsc.load_gather(list_d, [it, ov]).astype(_I32)
            sviv = jnp.where(validv, sviv, 0)
            dviv = jnp.where(validv, dviv, _bcast(base, _I32))
            idxbuf[pl.ds(j * 16, 16)] = sviv
            dbuf[pl.ds(j * 16, 16)] = dviv
        cp1 = pltpu.async_copy(h2x.at[idxbuf], rows, sem1)
        cp1.wait()

        for g in range(CGRP):
            it = _iota16()
            c16 = _splat_i32(16)
            m2v = plsc.load_gather(own, [jnp.zeros((16,), _I32),
                                         _splat_i32(18)])
            cntv = cntb[...]
            validv = _bcast(k * CGRP + g, _I32) < cntv
            dstv = dbuf[pl.ds(g * 16, 16)]
            dlocv = jnp.clip(dstv - _bcast(base, _I32), 0, BUCK - 1)
            rowids = it + _bcast(g * 16, _I32)
            as2v = plsc.load_gather(rows, [rowids, c16])
            ad2v = plsc.load_gather(own, [dlocv, _splat_i32(17)])
            e1 = as2v + ad2v
            e1 = jnp.maximum(e1, 0.2 * e1)
            w16 = jnp.exp(e1 - m2v)
            w16 = jnp.where(validv, w16, 0.0)
            for e in range(16):
                wspl = _take16(w16, _splat_i32(e))
                dspl = _take16(dlocv, _splat_i32(e))
                hv = rows[g * 16 + e, pl.ds(0, 16)]
                plsc.addupdate_scatter(acc2, [dspl, it], wspl * hv)
                plsc.addupdate_scatter(acc2, [dspl, c16 + it], wspl)
        return 0

    lax.fori_loop(0, nch, _chunk, 0)
    pltpu.sync_copy(acc2, accs2.at[wid])


def _sc_d(lsrc, ldst, counts, h2x):
    mesh = plsc.VectorSubcoreMesh(core_axis_name="c", subcore_axis_name="s")
    kern = pl.kernel(
        _sc_d_body,
        out_type=jax.ShapeDtypeStruct((NT, BUCK, ACC2_W), _F32),
        mesh=mesh,
        scratch_types=[
            pltpu.VMEM((16, LCAP), _F32),
            pltpu.VMEM((16, LCAP), _F32),
            pltpu.VMEM((BUCK, 128), _F32),
            pltpu.VMEM((C, 128), _F32),
            pltpu.VMEM((BUCK, ACC2_W), _F32),
            pltpu.VMEM((16,), _I32),
            pltpu.VMEM((16,), _I32),
            pltpu.VMEM((C,), _I32),
            pltpu.VMEM((C,), _I32),
            pltpu.SemaphoreType.DMA,
        ],
        compiler_params=pltpu.CompilerParams(needs_layout_passes=False),
    )
    return kern(lsrc, ldst, counts, h2x)


def _tc_e_body(a_ref, b2_ref, out_ref):
    a = a_ref[...]
    o = a[:, :16] / (a[:, 16:17] + 1e-16) + b2_ref[...]
    o = o - jnp.max(o, axis=1, keepdims=True)
    out_ref[...] = o - jnp.log(jnp.sum(jnp.exp(o), axis=1, keepdims=True))


def _tc_e(accs2_flat, b2r):
    return pl.pallas_call(
        _tc_e_body,
        out_shape=jax.ShapeDtypeStruct((NP, 16), _F32),
    )(accs2_flat, b2r)


@jax.jit
def kernel(x, edge_index, W1, a_s1, a_d1, b1, W2, a_s2, a_d2, b2):
    xp = jnp.pad(x, ((0, NP - N), (0, 0)))
    esrc = edge_index[0].astype(_I32)
    edst = edge_index[1].astype(_I32)

    e8 = jnp.repeat(jnp.eye(8, dtype=_F32), 16, axis=0)
    asw = e8 * a_s1.reshape(128, 1)
    adw = e8 * a_d1.reshape(128, 1)

    h1, aa1, m1 = _tc_a(xp, W1, asw, adw)
    accs, lsrc, ldst, counts = _sc_b(esrc, edst, h1, aa1, m1)

    h2x = _tc_c(
        accs.reshape(NP, ACC1_W),
        W2,
        e8.T,
        b1.reshape(1, 128),
        a_s2.reshape(1, 16),
        a_d2.reshape(1, 16),
    )
    accs2 = _sc_d(lsrc, ldst, counts, h2x)
    out = _tc_e(accs2.reshape(NP, ACC2_W), b2.reshape(1, 16))
    return out[:N]
